# Initial kernel scaffold; baseline (speedup 1.0000x reference)
import jax
import jax.numpy as jnp
from jax.experimental import pallas as pl


def kernel(char_tokens, char_embeddings, word_codebook, proj_W, proj_b):
    raise NotImplementedError("write your pallas kernel here")


# speedup vs baseline: 1.0118x; 1.0118x over previous
import jax
import jax.numpy as jnp
from jax.experimental import pallas as pl

_D = 64
_K = 1024
_WL = 2
_BLK = 1024


def _vq_kernel(x_ref, cbT_ref, cb_ref, pwT_ref, pb_ref,
               idx_ref, emb_ref, loss_ref):
    i = pl.program_id(0)
    nsteps = pl.num_programs(0)
    x = x_ref[...]
    we = (x[:, :_D] + x[:, _D:]) * 0.5
    we2 = jnp.sum(we * we, axis=1, keepdims=True)
    cb = cb_ref[...]
    cb2 = jnp.sum(cb * cb, axis=1)[None, :]
    scores = jax.lax.dot_general(
        we, cbT_ref[...], (((1,), (0,)), ((), ())),
        preferred_element_type=jnp.float32)
    d2 = we2 - 2.0 * scores + cb2
    dist = jnp.sqrt(jnp.maximum(d2, 0.0))
    lanes = jax.lax.broadcasted_iota(jnp.int32, (x.shape[0], _K), 1)
    mind = jnp.min(dist, axis=1, keepdims=True)
    at_min = dist == mind
    idx = jnp.min(jnp.where(at_min, lanes, _K), axis=1).astype(jnp.int32)
    idx_ref[0, 0, :] = idx
    onehot = (lanes == idx[:, None]).astype(jnp.float32)
    emb = jax.lax.dot_general(
        onehot, cb, (((1,), (0,)), ((), ())),
        preferred_element_type=jnp.float32,
        precision=jax.lax.Precision.HIGHEST)
    emb_ref[...] = emb
    proj = jax.lax.dot_general(
        emb, pwT_ref[...], (((1,), (0,)), ((), ())),
        preferred_element_type=jnp.float32) + pb_ref[...]
    r = proj - emb
    part = jnp.sum(r * r)

    @pl.when(i == 0)
    def _init():
        loss_ref[...] = jnp.zeros_like(loss_ref)

    loss_ref[...] += part.reshape(1, 1)

    @pl.when(i == nsteps - 1)
    def _finalize():
        loss_ref[...] = loss_ref[...] / (nsteps * _BLK * _D)


def kernel(char_tokens, char_embeddings, word_codebook, proj_W, proj_b):
    Bv, Lv, Dv = char_embeddings.shape
    nw = Lv // _WL
    total = Bv * nw
    x = char_embeddings.reshape(total, _WL * Dv)
    cbT = word_codebook.T
    pwT = proj_W.T
    pb = proj_b.reshape(1, Dv)
    grid = total // _BLK
    idx3, emb, loss = pl.pallas_call(
        _vq_kernel,
        grid=(grid,),
        in_specs=[
            pl.BlockSpec((_BLK, _WL * Dv), lambda i: (i, 0)),
            pl.BlockSpec((Dv, _K), lambda i: (0, 0)),
            pl.BlockSpec((_K, Dv), lambda i: (0, 0)),
            pl.BlockSpec((Dv, Dv), lambda i: (0, 0)),
            pl.BlockSpec((1, Dv), lambda i: (0, 0)),
        ],
        out_specs=[
            pl.BlockSpec((1, 1, _BLK), lambda i: (i, 0, 0)),
            pl.BlockSpec((_BLK, Dv), lambda i: (i, 0)),
            pl.BlockSpec((1, 1), lambda i: (0, 0)),
        ],
        out_shape=[
            jax.ShapeDtypeStruct((grid, 1, _BLK), jnp.int32),
            jax.ShapeDtypeStruct((total, Dv), jnp.float32),
            jax.ShapeDtypeStruct((1, 1), jnp.float32),
        ],
    )(x, cbT, word_codebook, pwT, pb)
    word_indices = idx3.reshape(Bv, nw)
    word_embeddings = emb.reshape(Bv, nw, Dv)
    return word_indices, word_embeddings, loss[0, 0]


# speedup vs baseline: 1.1072x; 1.0943x over previous
import jax
import jax.numpy as jnp
from jax.experimental import pallas as pl
from jax.experimental.pallas import tpu as pltpu

_D = 64
_K = 1024
_WL = 2
_BLK = 256


def _vq_kernel(x_ref, cbT_ref, cb_ref, pwT_ref, pb_ref,
               idx_ref, emb_ref, loss_ref, cb2_s, cbhi_s, cblo_s):
    i = pl.program_id(0)
    ns = pl.num_programs(0)

    @pl.when(i == 0)
    def _prep():
        cb = cb_ref[...]
        cb2_s[...] = jnp.sum(cb * cb, axis=1)[None, :]
        hi = cb.astype(jnp.bfloat16).astype(jnp.float32)
        cbhi_s[...] = hi
        cblo_s[...] = cb - hi
        loss_ref[...] = jnp.zeros_like(loss_ref)

    x = x_ref[...]
    we = (x[:, :_D] + x[:, _D:]) * 0.5
    we2 = jnp.sum(we * we, axis=1, keepdims=True)
    scores = jax.lax.dot_general(
        we, cbT_ref[...], (((1,), (0,)), ((), ())),
        preferred_element_type=jnp.float32)
    dist = jnp.sqrt(jnp.maximum(we2 - 2.0 * scores + cb2_s[...], 0.0))
    mind = jnp.min(dist, axis=1, keepdims=True)
    lanesf = jax.lax.broadcasted_iota(jnp.int32, dist.shape, 1).astype(jnp.float32)
    idxf = jnp.min(jnp.where(dist == mind, lanesf, float(_K)), axis=1)
    idx_ref[0, 0, :] = idxf.astype(jnp.int32)
    onehot = (lanesf == idxf[:, None]).astype(jnp.float32)
    emb = (jax.lax.dot_general(onehot, cbhi_s[...], (((1,), (0,)), ((), ())),
                               preferred_element_type=jnp.float32)
           + jax.lax.dot_general(onehot, cblo_s[...], (((1,), (0,)), ((), ())),
                                 preferred_element_type=jnp.float32))
    emb_ref[...] = emb
    proj = jax.lax.dot_general(
        emb, pwT_ref[...], (((1,), (0,)), ((), ())),
        preferred_element_type=jnp.float32) + pb_ref[...]
    r = proj - emb
    loss_ref[...] += jnp.sum(r * r).reshape(1, 1)

    @pl.when(i == ns - 1)
    def _fin():
        loss_ref[...] = loss_ref[...] / (ns * _BLK * _D)


def kernel(char_tokens, char_embeddings, word_codebook, proj_W, proj_b):
    Bv, Lv, Dv = char_embeddings.shape
    nw = Lv // _WL
    total = Bv * nw
    x = char_embeddings.reshape(total, _WL * Dv)
    cbT = word_codebook.T
    pwT = proj_W.T
    pb = proj_b.reshape(1, Dv)
    grid = total // _BLK
    idx3, emb, loss = pl.pallas_call(
        _vq_kernel,
        grid=(grid,),
        in_specs=[
            pl.BlockSpec((_BLK, _WL * Dv), lambda i: (i, 0)),
            pl.BlockSpec((Dv, _K), lambda i: (0, 0)),
            pl.BlockSpec((_K, Dv), lambda i: (0, 0)),
            pl.BlockSpec((Dv, Dv), lambda i: (0, 0)),
            pl.BlockSpec((1, Dv), lambda i: (0, 0)),
        ],
        out_specs=[
            pl.BlockSpec((1, 1, _BLK), lambda i: (i, 0, 0)),
            pl.BlockSpec((_BLK, Dv), lambda i: (i, 0)),
            pl.BlockSpec((1, 1), lambda i: (0, 0)),
        ],
        out_shape=[
            jax.ShapeDtypeStruct((grid, 1, _BLK), jnp.int32),
            jax.ShapeDtypeStruct((total, Dv), jnp.float32),
            jax.ShapeDtypeStruct((1, 1), jnp.float32),
        ],
        scratch_shapes=[
            pltpu.VMEM((1, _K), jnp.float32),
            pltpu.VMEM((_K, _D), jnp.float32),
            pltpu.VMEM((_K, _D), jnp.float32),
        ],
    )(x, cbT, word_codebook, pwT, pb)
    word_indices = idx3.reshape(Bv, nw)
    word_embeddings = emb.reshape(Bv, nw, Dv)
    return word_indices, word_embeddings, loss[0, 0]


# speedup vs baseline: 1.4178x; 1.2805x over previous
import jax
import jax.numpy as jnp
from jax.experimental import pallas as pl
from jax.experimental.pallas import tpu as pltpu

_D = 64
_K = 1024
_WL = 2
_BLK = 4096


def _vq_kernel(x_ref, cbT2_ref, cb_ref, pwT_ref, pb_ref,
               idx_ref, emb_ref, loss_ref, cb2_s, lanes_s, cbhi_s, cblo_s):
    i = pl.program_id(0)
    ns = pl.num_programs(0)

    @pl.when(i == 0)
    def _prep():
        cb = cb_ref[...]
        cb2_s[...] = jnp.sum(cb * cb, axis=1)[None, :]
        lanes_s[...] = jax.lax.broadcasted_iota(
            jnp.int32, (1, _K), 1).astype(jnp.float32)
        hi = cb.astype(jnp.bfloat16)
        cbhi_s[...] = hi
        cblo_s[...] = (cb - hi.astype(jnp.float32)).astype(jnp.bfloat16)
        loss_ref[...] = jnp.zeros_like(loss_ref)

    x = x_ref[...]
    we = (x[:, :_D] + x[:, _D:]) * 0.5
    we2 = jnp.sum(we * we, axis=1, keepdims=True)
    sc2 = jax.lax.dot_general(
        we, cbT2_ref[...], (((1,), (0,)), ((), ())),
        preferred_element_type=jnp.float32)
    d2 = (we2 - sc2) + cb2_s[...]
    dist = jnp.sqrt(jnp.maximum(d2, 0.0))
    mind = jnp.min(dist, axis=1, keepdims=True)
    at_min = dist == mind
    lanesf = lanes_s[...]
    idxf = jnp.min(jnp.where(at_min, lanesf, float(_K)), axis=1)
    idx_ref[...] = idxf.astype(jnp.int32)[:, None]
    onehot = (lanesf == idxf[:, None]).astype(jnp.bfloat16)
    emb = (jax.lax.dot_general(onehot, cbhi_s[...], (((1,), (0,)), ((), ())),
                               preferred_element_type=jnp.float32)
           + jax.lax.dot_general(onehot, cblo_s[...], (((1,), (0,)), ((), ())),
                                 preferred_element_type=jnp.float32))
    emb_ref[...] = emb
    proj = jax.lax.dot_general(
        emb, pwT_ref[...], (((1,), (0,)), ((), ())),
        preferred_element_type=jnp.float32) + pb_ref[...]
    rr = proj - emb
    loss_ref[...] += jnp.sum(rr * rr).reshape(1, 1)

    @pl.when(i == ns - 1)
    def _fin():
        loss_ref[...] = loss_ref[...] / (ns * _BLK * _D)


def kernel(char_tokens, char_embeddings, word_codebook, proj_W, proj_b):
    Bv, Lv, Dv = char_embeddings.shape
    nw = Lv // _WL
    total = Bv * nw
    x = char_embeddings.reshape(total, _WL * Dv)
    cbT2 = word_codebook.T * 2.0
    pwT = proj_W.T
    pb = proj_b.reshape(1, Dv)
    grid = total // _BLK
    idx2, emb, loss = pl.pallas_call(
        _vq_kernel,
        grid=(grid,),
        in_specs=[
            pl.BlockSpec((_BLK, _WL * Dv), lambda i: (i, 0)),
            pl.BlockSpec((Dv, _K), lambda i: (0, 0)),
            pl.BlockSpec((_K, Dv), lambda i: (0, 0)),
            pl.BlockSpec((Dv, Dv), lambda i: (0, 0)),
            pl.BlockSpec((1, Dv), lambda i: (0, 0)),
        ],
        out_specs=[
            pl.BlockSpec((_BLK, 1), lambda i: (i, 0)),
            pl.BlockSpec((_BLK, Dv), lambda i: (i, 0)),
            pl.BlockSpec((1, 1), lambda i: (0, 0)),
        ],
        out_shape=[
            jax.ShapeDtypeStruct((total, 1), jnp.int32),
            jax.ShapeDtypeStruct((total, Dv), jnp.float32),
            jax.ShapeDtypeStruct((1, 1), jnp.float32),
        ],
        scratch_shapes=[
            pltpu.VMEM((1, _K), jnp.float32),
            pltpu.VMEM((1, _K), jnp.float32),
            pltpu.VMEM((_K, _D), jnp.bfloat16),
            pltpu.VMEM((_K, _D), jnp.bfloat16),
        ],
    )(x, cbT2, word_codebook, pwT, pb)
    word_indices = idx2.reshape(Bv, nw)
    word_embeddings = emb.reshape(Bv, nw, Dv)
    return word_indices, word_embeddings, loss[0, 0]
